# baseline (speedup 1.0000x reference)
import jax
import jax.numpy as jnp
from jax import lax
from jax.experimental import pallas as pl
from jax.experimental.pallas import tpu as pltpu
from jax.experimental.pallas import tpu_sc as plsc

N_ROWS = 10000
D = 128
NUM_GRAPHS = 64
NC, NS = 2, 16
NW = NC * NS
ROWS_PER = 320
TAIL = N_ROWS - ROWS_PER * (NW - 1)
CH = 80
NCH = ROWS_PER // CH
GPC = CH // 16


def _sc_body(x_hbm, b_hbm, z_hbm, iota8_hbm, ps_hbm, pcc_hbm,
             x_v, ids_v, idn_v, cnt_v, acc_sh, cnt_sh, sem, xsem):
    cid = lax.axis_index("c")
    sid = lax.axis_index("s")
    wid = sid * NC + cid
    base = wid * ROWS_PER
    is_tail = wid == NW - 1

    @pl.when(sid == 0)
    def _():
        pltpu.sync_copy(z_hbm.at[pl.ds(0, NUM_GRAPHS)], acc_sh)
        pltpu.sync_copy(z_hbm.at[pl.ds(NUM_GRAPHS, 8)], cnt_sh)

    pltpu.sync_copy(iota8_hbm, idn_v)

    @pl.when(jnp.logical_not(is_tail))
    def _():
        for j in range(NCH):
            pltpu.sync_copy(b_hbm.at[pl.ds(base + j * CH, CH)], ids_v.at[j])
        for j in range(NCH):
            pltpu.async_copy(x_hbm.at[pl.ds(base + j * CH, CH)],
                             x_v.at[pl.ds(j * CH, CH)], xsem)

    @pl.when(is_tail)
    def _():
        pltpu.sync_copy(b_hbm.at[pl.ds(base, TAIL)], ids_v.at[0])
        pltpu.async_copy(x_hbm.at[pl.ds(base, TAIL)],
                         x_v.at[pl.ds(0, TAIL)], xsem)

    zero16 = jnp.zeros((16,), jnp.float32)
    ones16 = jnp.ones((16,), jnp.float32)

    def zb(i, c):
        for k in range(8):
            cnt_v[i, pl.ds(k * 16, 16)] = zero16
        return c

    lax.fori_loop(0, 8, zb, 0)

    def grp(g, c):
        idv = ids_v[g // GPC, pl.ds((g % GPC) * 16, 16)]
        for l in range(16):
            s = idv[l]
            plsc.addupdate(cnt_v.at[s // 8, pl.ds((s % 8) * 16, 16)], ones16)
        return c

    ngrp = jnp.where(is_tail, TAIL // 16, ROWS_PER // 16)
    lax.fori_loop(0, ngrp, grp, 0)

    plsc.subcore_barrier()

    @pl.when(jnp.logical_not(is_tail))
    def _():
        for j in range(NCH):
            pltpu.make_async_copy(x_hbm.at[pl.ds(base + j * CH, CH)],
                                  x_v.at[pl.ds(j * CH, CH)], xsem).wait()
            pltpu.async_copy(x_v.at[pl.ds(j * CH, CH)],
                             acc_sh.at[ids_v.at[j]], sem, add=True)
        for j in range(NCH):
            pltpu.make_async_copy(x_v.at[pl.ds(j * CH, CH)],
                                  acc_sh.at[ids_v.at[j]], sem).wait()

    @pl.when(is_tail)
    def _():
        pltpu.make_async_copy(x_hbm.at[pl.ds(base, TAIL)],
                              x_v.at[pl.ds(0, TAIL)], xsem).wait()
        pltpu.async_copy(x_v.at[pl.ds(0, TAIL)],
                         acc_sh.at[ids_v.at[0]], sem, add=True)
        pltpu.make_async_copy(x_v.at[pl.ds(0, TAIL)],
                              acc_sh.at[ids_v.at[0]], sem).wait()

    pltpu.sync_copy(cnt_v, cnt_sh.at[idn_v.at[0]], add=True)

    plsc.subcore_barrier()

    @pl.when(sid == 0)
    def _():
        pltpu.sync_copy(acc_sh, ps_hbm.at[cid])
        pltpu.sync_copy(cnt_sh, pcc_hbm.at[cid])


@jax.jit
def _segment_partials(x, batch_i32, z, iota8):
    mesh = plsc.VectorSubcoreMesh(core_axis_name="c", subcore_axis_name="s",
                                  num_cores=NC, num_subcores=NS)
    f = pl.kernel(
        _sc_body,
        out_type=(
            jax.ShapeDtypeStruct((NC, NUM_GRAPHS, D), jnp.float32),
            jax.ShapeDtypeStruct((NC, 8, D), jnp.float32),
        ),
        mesh=mesh,
        scratch_types=[
            pltpu.VMEM((ROWS_PER, D), jnp.float32),
            pltpu.VMEM((NCH, CH), jnp.int32),
            pltpu.VMEM((1, 8), jnp.int32),
            pltpu.VMEM((8, D), jnp.float32),
            pltpu.VMEM_SHARED((NUM_GRAPHS, D), jnp.float32),
            pltpu.VMEM_SHARED((8, D), jnp.float32),
            pltpu.SemaphoreType.DMA,
            pltpu.SemaphoreType.DMA,
        ],
    )
    return f(x, batch_i32, z, iota8)


def _tc_body(ps_ref, pcc_ref, u_ref, w1_ref, b1_ref, w2_ref, b2_ref, y_ref):
    sums = ps_ref[0] + ps_ref[1]
    c2 = pcc_ref[0] + pcc_ref[1]
    lane = lax.broadcasted_iota(jnp.int32, (D, 8), 0)
    sel = (lane == lax.broadcasted_iota(jnp.int32, (D, 8), 1) * 16)
    c88 = c2 @ sel.astype(jnp.float32)
    srow = lax.broadcasted_iota(jnp.int32, (NUM_GRAPHS, 8), 0) // 8
    pick = (srow == lax.broadcasted_iota(jnp.int32, (NUM_GRAPHS, 8), 1))
    c648 = pick.astype(jnp.float32) @ c88
    scol = lax.broadcasted_iota(jnp.int32, (NUM_GRAPHS, 8), 0) % 8
    mask = (scol == lax.broadcasted_iota(jnp.int32, (NUM_GRAPHS, 8), 1))
    cnt = jnp.sum(jnp.where(mask, c648, 0.0), axis=1, keepdims=True)
    agg = sums / jnp.maximum(cnt, 1.0)
    out = jnp.concatenate([u_ref[...], agg], axis=1)
    h = jnp.maximum(out @ w1_ref[...] + b1_ref[...], 0.0)
    y_ref[...] = h @ w2_ref[...] + b2_ref[...]


@jax.jit
def _pool_mlp(ps, pcc, u, W1, b1, W2, b2):
    return pl.pallas_call(
        _tc_body,
        out_shape=jax.ShapeDtypeStruct((NUM_GRAPHS, 32), jnp.float32),
    )(ps, pcc, u, W1, b1.reshape(1, 32), W2, b2.reshape(1, 32))


def kernel(x, edge_index, edge_attr, u, batch, W1, b1, W2, b2):
    del edge_index, edge_attr
    batch_i32 = batch.astype(jnp.int32)
    z = jnp.zeros((NUM_GRAPHS + 8, D), jnp.float32)
    iota8 = jnp.arange(8, dtype=jnp.int32).reshape(1, 8)
    ps, pcc = _segment_partials(x, batch_i32, z, iota8)
    return _pool_mlp(ps, pcc, u, W1, b1, W2, b2)
